# Initial kernel scaffold; baseline (speedup 1.0000x reference)
import jax
import jax.numpy as jnp
from jax.experimental import pallas as pl

_B = 2048
_D = 256
_E = 16
_BLK = 128
_NBLK = _B // _BLK


def _body(a_ref, x_ref, w_ref, b_ref, o_ref):
    i = pl.program_id(0)
    av = a_ref[i]
    x = x_ref[...]
    acc = jnp.zeros((_BLK, _D), jnp.float32)
    for e in range(_E):
        y = jax.lax.dot_general(
            x, w_ref[e],
            (((1,), (1,)), ((), ())),
            preferred_element_type=jnp.float32,
        )
        y = y + b_ref[e][None, :]
        m = (av == e)[:, None]
        acc = jnp.where(m, y, acc)
    o_ref[...] = acc


def kernel(x, a, weight, bias):
    a2 = a.astype(jnp.int32).reshape(_NBLK, _BLK)
    xb = x.astype(jnp.bfloat16)
    wb = weight.astype(jnp.bfloat16)
    return pl.pallas_call(
        _body,
        grid=(_NBLK,),
        in_specs=[
            pl.BlockSpec((_NBLK, _BLK), lambda i: (0, 0)),
            pl.BlockSpec((_BLK, _D), lambda i: (i, 0)),
            pl.BlockSpec((_E, _D, _D), lambda i: (0, 0, 0)),
            pl.BlockSpec((_E, _D), lambda i: (0, 0)),
        ],
        out_specs=pl.BlockSpec((_BLK, _D), lambda i: (i, 0)),
        out_shape=jax.ShapeDtypeStruct((_B, _D), jnp.float32),
    )(a2, xb, wb, bias)


# speedup vs baseline: 8.8443x; 8.8443x over previous
import jax
import jax.numpy as jnp
from jax.experimental import pallas as pl

_B = 2048
_D = 256
_E = 16
_BLK = 128
_NBLK = _B // _BLK


def _body(a_ref, x_ref, w_ref, b_ref, o_ref):
    av = a_ref[...]
    x = x_ref[...]
    acc = jnp.zeros((_BLK, _D), jnp.float32)
    for e in range(_E):
        y = jax.lax.dot_general(
            x, w_ref[e],
            (((1,), (1,)), ((), ())),
            preferred_element_type=jnp.float32,
        )
        y = y + b_ref[e][None, :]
        m = av == e
        acc = jnp.where(m, y, acc)
    o_ref[...] = acc


def kernel(x, a, weight, bias):
    a2 = a.astype(jnp.int32).reshape(_B, 1)
    xb = x.astype(jnp.bfloat16)
    wb = weight.astype(jnp.bfloat16)
    return pl.pallas_call(
        _body,
        grid=(_NBLK,),
        in_specs=[
            pl.BlockSpec((_BLK, 1), lambda i: (i, 0)),
            pl.BlockSpec((_BLK, _D), lambda i: (i, 0)),
            pl.BlockSpec((_E, _D, _D), lambda i: (0, 0, 0)),
            pl.BlockSpec((_E, _D), lambda i: (0, 0)),
        ],
        out_specs=pl.BlockSpec((_BLK, _D), lambda i: (i, 0)),
        out_shape=jax.ShapeDtypeStruct((_B, _D), jnp.float32),
    )(a2, xb, wb, bias)
